# baseline (speedup 1.0000x reference)
import jax
import jax.numpy as jnp
from jax import lax
from jax.experimental import pallas as pl
from jax.experimental.pallas import tpu as pltpu

_B = 128
_SPAD = 256
_TOKENS = _B * (_B - 1) // 2
_D = 1024
_BLK = 4064


def _fused_body(values_ref, rl_ref, vout_ref, splits_ref):
    vout_ref[...] = values_ref[...]

    @pl.when(pl.program_id(0) == 0)
    def _():
        rl_row = rl_ref[...].astype(jnp.float32)
        j = lax.broadcasted_iota(jnp.int32, (_B, _SPAD), 0)
        i = lax.broadcasted_iota(jnp.int32, (_B, _SPAD), 1)
        tri = jnp.where(j < i, 1.0, 0.0)
        splits = jnp.dot(rl_row, tri, preferred_element_type=jnp.float32)
        splits_ref[...] = splits.astype(jnp.int32)


_fused_tc = pl.pallas_call(
    _fused_body,
    grid=(_TOKENS // _BLK,),
    in_specs=[
        pl.BlockSpec((_BLK, _D), lambda i: (i, 0)),
        pl.BlockSpec((1, _B), lambda i: (0, 0)),
    ],
    out_specs=[
        pl.BlockSpec((_BLK, _D), lambda i: (i, 0)),
        pl.BlockSpec((1, _SPAD), lambda i: (0, 0)),
    ],
    out_shape=[
        jax.ShapeDtypeStruct((_TOKENS, _D), jnp.float32),
        jax.ShapeDtypeStruct((1, _SPAD), jnp.int32),
    ],
    compiler_params=pltpu.CompilerParams(vmem_limit_bytes=100 * 1024 * 1024),
)


def kernel(values, row_lengths):
    values_out, splits_pad = _fused_tc(values, row_lengths.reshape(1, _B))
    row_splits = splits_pad.reshape(_SPAD)[: _B + 1]
    return values_out, row_splits
